# baseline (speedup 1.0000x reference)
import functools

import jax
import jax.numpy as jnp
from jax import lax
from jax.experimental import pallas as pl
from jax.experimental.pallas import tpu as pltpu
from jax.experimental.pallas import tpu_sc as plsc

_N, _E, _D = 10000, 320000, 128
_NC, _NS = 2, 16
_NW = _NC * _NS
_EPW = _E // _NW
_K = 80
_NCHUNK = _EPW // _K
_RPT = 624
_RTAIL = _N - _RPT * _NS
_VL = 16

_mesh = plsc.VectorSubcoreMesh(core_axis_name="c", subcore_axis_name="s")


@functools.partial(
    pl.kernel,
    out_type=jax.ShapeDtypeStruct((_NC, _N, _D), jnp.float32),
    mesh=_mesh,
    scratch_types=[
        pltpu.VMEM((_K,), jnp.int32),
        pltpu.VMEM((_K,), jnp.int32),
        pltpu.VMEM((_K,), jnp.int32),
        pltpu.VMEM((_K,), jnp.int32),
        pltpu.VMEM((_K,), jnp.float32),
        pltpu.VMEM((_K,), jnp.float32),
        pltpu.VMEM((_K, _D), jnp.float32),
        pltpu.VMEM((_K, _D), jnp.float32),
        pltpu.VMEM((_K, _D), jnp.float32),
        pltpu.VMEM((_K, _D), jnp.float32),
        pltpu.VMEM_SHARED((_N, _D), jnp.float32),
        pltpu.SemaphoreType.DMA,
        pltpu.SemaphoreType.DMA,
        pltpu.SemaphoreType.DMA,
        pltpu.SemaphoreType.DMA,
        pltpu.SemaphoreType.DMA,
        pltpu.SemaphoreType.DMA,
        pltpu.SemaphoreType.DMA,
        pltpu.SemaphoreType.DMA,
        pltpu.SemaphoreType.DMA,
        pltpu.SemaphoreType.DMA,
    ],
)
def _sc_conv(h_hbm, s_hbm, r_hbm, a_hbm, out_hbm,
             sx0, sx1, rx0, rx1, av0, av1, gb0, gb1, sb0, sb1, acc,
             gsem0, gsem1, ssem0, ssem1, xsem0, xsem1, rsem0, rsem1,
             asem0, asem1):
    cid = lax.axis_index("c")
    sid = lax.axis_index("s")
    wid = sid * _NC + cid
    ebase = wid * _EPW

    def _zfill(i, carry):
        r = i // (_D // _VL)
        j = lax.rem(i, _D // _VL)
        sb0[r, pl.ds(j * _VL, _VL)] = jnp.zeros((_VL,), jnp.float32)
        return carry

    lax.fori_loop(0, _K * (_D // _VL), _zfill, None)
    full, rem = _RPT // _K, _RPT % _K
    for k in range(full):
        pltpu.sync_copy(sb0, acc.at[pl.ds(sid * _RPT + k * _K, _K)])
    if rem:
        pltpu.sync_copy(sb0.at[pl.ds(0, rem)],
                        acc.at[pl.ds(sid * _RPT + full * _K, rem)])

    @pl.when(sid == _NS - 1)
    def _zero_tail():
        pltpu.sync_copy(sb0.at[pl.ds(0, _RTAIL)],
                        acc.at[pl.ds(_RPT * _NS, _RTAIL)])

    plsc.subcore_barrier()

    def _scale(av, gb, sb):
        @plsc.parallel_loop(0, _K // _VL, unroll=2)
        def _body(eb):
            a16 = av[pl.ds(eb * _VL, _VL)]
            for i in range(_VL):
                e = eb * _VL + i
                a = a16[i]
                for j in range(_D // _VL):
                    sl = pl.ds(j * _VL, _VL)
                    sb[e, sl] = gb[e, sl] * a

    def _edge_src(arr, c):
        return arr.at[pl.ds(ebase + pl.multiple_of(c * _K, _K), _K)]

    slots = ((gb0, sb0, sx0, rx0, av0, gsem0, ssem0, xsem0, rsem0, asem0),
             (gb1, sb1, sx1, rx1, av1, gsem1, ssem1, xsem1, rsem1, asem1))

    for b, (gb, sb, sx, rx, av, gsem, ssem, xsem, rsem, asem) in enumerate(slots):
        pltpu.sync_copy(_edge_src(s_hbm, b), sx)
        pltpu.async_copy(_edge_src(a_hbm, b), av, asem)
        pltpu.async_copy(h_hbm.at[sx], gb, gsem)

    def _group(g, carry):
        for b, (gb, sb, sx, rx, av, gsem, ssem, xsem, rsem, asem) in enumerate(slots):
            c = g * 2 + b
            has_next = (c + 2 <= _NCHUNK - 1) if b == 0 else None

            pltpu.make_async_copy(h_hbm.at[sx], gb, gsem).wait()

            @pl.when(g >= 1)
            def _drain_prev():
                pltpu.make_async_copy(sb, acc.at[rx], ssem).wait()

            pltpu.async_copy(_edge_src(r_hbm, c), rx, rsem)

            def _fetch_next():
                pltpu.async_copy(_edge_src(s_hbm, c + 2), sx, xsem)

            if b == 0:
                _fetch_next()
            else:
                pl.when(g <= (_NCHUNK - 4) // 2)(_fetch_next)

            pltpu.make_async_copy(_edge_src(a_hbm, c), av, asem).wait()
            _scale(av, gb, sb)

            def _fetch_next_attr():
                pltpu.async_copy(_edge_src(a_hbm, c + 2), av, asem)

            if b == 0:
                _fetch_next_attr()
            else:
                pl.when(g <= (_NCHUNK - 4) // 2)(_fetch_next_attr)

            pltpu.make_async_copy(_edge_src(r_hbm, c), rx, rsem).wait()
            pltpu.async_copy(sb, acc.at[rx], ssem, add=True)

            def _start_next_gather():
                pltpu.make_async_copy(_edge_src(s_hbm, c + 2), sx, xsem).wait()
                pltpu.async_copy(h_hbm.at[sx], gb, gsem)

            if b == 0:
                _start_next_gather()
            else:
                pl.when(g <= (_NCHUNK - 4) // 2)(_start_next_gather)
        return carry

    _ngroup = (_NCHUNK - 1) // 2
    lax.fori_loop(0, _ngroup, _group, None)

    tail = tuple(zip(range(2 * _ngroup, _NCHUNK), slots))
    for ct, (gb, sb, sx, rx, av, gsem, ssem, xsem, rsem, asem) in tail:
        pltpu.make_async_copy(h_hbm.at[sx], gb, gsem).wait()
        pltpu.make_async_copy(sb, acc.at[rx], ssem).wait()
        pltpu.sync_copy(_edge_src(r_hbm, ct), rx)
        pltpu.make_async_copy(_edge_src(a_hbm, ct), av, asem).wait()
        _scale(av, gb, sb)
        pltpu.async_copy(sb, acc.at[rx], ssem, add=True)
    for ct, (gb, sb, sx, rx, av, gsem, ssem, xsem, rsem, asem) in tail:
        pltpu.make_async_copy(sb, acc.at[rx], ssem).wait()
    for b, (gb, sb, sx, rx, av, gsem, ssem, xsem, rsem, asem) in enumerate(slots):
        if b >= _NCHUNK - 2 * _ngroup:
            pltpu.make_async_copy(sb, acc.at[rx], ssem).wait()

    plsc.subcore_barrier()
    pltpu.sync_copy(acc.at[pl.ds(sid * _RPT, _RPT)],
                    out_hbm.at[cid, pl.ds(sid * _RPT, _RPT)])

    @pl.when(sid == _NS - 1)
    def _write_tail():
        pltpu.sync_copy(acc.at[pl.ds(_RPT * _NS, _RTAIL)],
                        out_hbm.at[cid, pl.ds(_RPT * _NS, _RTAIL)])


_BN = 2000
_NB = _N // _BN


def _enc_body(x_ref, w0_ref, b0_ref, w1_ref, b1_ref, o_ref):
    h = jnp.dot(x_ref[...], w0_ref[...], preferred_element_type=jnp.float32)
    h = jnp.maximum(h + b0_ref[...], 0.0)
    h = jnp.dot(h, w1_ref[...], preferred_element_type=jnp.float32)
    o_ref[...] = jnp.maximum(h + b1_ref[...], 0.0)


def _hop_body(p_ref, w_ref, b_ref, o_ref):
    conv = p_ref[0] + p_ref[1]
    u = jnp.dot(conv, w_ref[...], preferred_element_type=jnp.float32)
    o_ref[...] = jnp.maximum(u + b_ref[...], 0.0) + conv


def _dec_body(h_ref, w0_ref, b0_ref, w1_ref, b1_ref, o_ref):
    u = jnp.dot(h_ref[...], w0_ref[...], preferred_element_type=jnp.float32)
    u = jnp.maximum(u + b0_ref[...], 0.0)
    u = jnp.dot(u, w1_ref[...], preferred_element_type=jnp.float32)
    o_ref[...] = u + b1_ref[...]


def _row_blocked(body, n_weight_args):
    w_specs = [pl.BlockSpec((_D, _D), lambda i: (0, 0)),
               pl.BlockSpec((1, _D), lambda i: (0, 0))] * n_weight_args
    return pl.pallas_call(
        body,
        grid=(_NB,),
        in_specs=[pl.BlockSpec((_BN, _D), lambda i: (i, 0))] + w_specs,
        out_specs=pl.BlockSpec((_BN, _D), lambda i: (i, 0)),
        out_shape=jax.ShapeDtypeStruct((_N, _D), jnp.float32),
    )


_encoder = _row_blocked(_enc_body, 2)
_decoder = _row_blocked(_dec_body, 2)

_hop = pl.pallas_call(
    _hop_body,
    grid=(_NB,),
    in_specs=[pl.BlockSpec((_NC, _BN, _D), lambda i: (0, i, 0)),
              pl.BlockSpec((_D, _D), lambda i: (0, 0)),
              pl.BlockSpec((1, _D), lambda i: (0, 0))],
    out_specs=pl.BlockSpec((_BN, _D), lambda i: (i, 0)),
    out_shape=jax.ShapeDtypeStruct((_N, _D), jnp.float32),
)


def kernel(x, edge_index, edge_attr, enc_W0, enc_b0, enc_W1, enc_b1,
           hop0_W, hop0_b, hop1_W, hop1_b, hop2_W, hop2_b,
           dec_W0, dec_b0, dec_W1, dec_b1):
    senders = edge_index[1]
    receivers = edge_index[0]
    attr = edge_attr[:, 0]

    h = _encoder(x, enc_W0, enc_b0.reshape(1, _D), enc_W1, enc_b1.reshape(1, _D))
    for W, b in ((hop0_W, hop0_b), (hop1_W, hop1_b), (hop2_W, hop2_b)):
        partials = _sc_conv(h, senders, receivers, attr)
        h = _hop(partials, W, b.reshape(1, _D))

    c = dec_W1.shape[1]
    w1p = jnp.pad(dec_W1, ((0, 0), (0, _D - c)))
    b1p = jnp.pad(dec_b1, (0, _D - c)).reshape(1, _D)
    out = _decoder(h, dec_W0, dec_b0.reshape(1, _D), w1p, b1p)
    return out[:, :c]


# speedup vs baseline: 1.0051x; 1.0051x over previous
import functools

import jax
import jax.numpy as jnp
from jax import lax
from jax.experimental import pallas as pl
from jax.experimental.pallas import tpu as pltpu
from jax.experimental.pallas import tpu_sc as plsc

_N, _E, _D = 10000, 320000, 128
_NC, _NS = 2, 16
_NW = _NC * _NS
_EPW = _E // _NW
_K = 80
_NCHUNK = _EPW // _K
_RPT = 624
_RTAIL = _N - _RPT * _NS
_VL = 16

_mesh = plsc.VectorSubcoreMesh(core_axis_name="c", subcore_axis_name="s")


@functools.partial(
    pl.kernel,
    out_type=jax.ShapeDtypeStruct((_NC, _N, _D), jnp.float32),
    mesh=_mesh,
    scratch_types=[
        pltpu.VMEM((_K,), jnp.int32),
        pltpu.VMEM((_K,), jnp.int32),
        pltpu.VMEM((_K,), jnp.int32),
        pltpu.VMEM((_K,), jnp.int32),
        pltpu.VMEM((_K,), jnp.float32),
        pltpu.VMEM((_K,), jnp.float32),
        pltpu.VMEM((_K, _D), jnp.float32),
        pltpu.VMEM((_K, _D), jnp.float32),
        pltpu.VMEM((_K, _D), jnp.float32),
        pltpu.VMEM((_K, _D), jnp.float32),
        pltpu.VMEM_SHARED((_N, _D), jnp.float32),
        pltpu.SemaphoreType.DMA,
        pltpu.SemaphoreType.DMA,
        pltpu.SemaphoreType.DMA,
        pltpu.SemaphoreType.DMA,
        pltpu.SemaphoreType.DMA,
        pltpu.SemaphoreType.DMA,
        pltpu.SemaphoreType.DMA,
        pltpu.SemaphoreType.DMA,
        pltpu.SemaphoreType.DMA,
        pltpu.SemaphoreType.DMA,
    ],
)
def _sc_conv(h_hbm, s_hbm, r_hbm, a_hbm, out_hbm,
             sx0, sx1, rx0, rx1, av0, av1, gb0, gb1, sb0, sb1, acc,
             gsem0, gsem1, ssem0, ssem1, xsem0, xsem1, rsem0, rsem1,
             asem0, asem1):
    cid = lax.axis_index("c")
    sid = lax.axis_index("s")
    wid = sid * _NC + cid
    ebase = wid * _EPW

    def _zfill(i, carry):
        r = i // (_D // _VL)
        j = lax.rem(i, _D // _VL)
        sb0[r, pl.ds(j * _VL, _VL)] = jnp.zeros((_VL,), jnp.float32)
        return carry

    lax.fori_loop(0, _K * (_D // _VL), _zfill, None)
    full, rem = _RPT // _K, _RPT % _K
    for k in range(full):
        pltpu.sync_copy(sb0, acc.at[pl.ds(sid * _RPT + k * _K, _K)])
    if rem:
        pltpu.sync_copy(sb0.at[pl.ds(0, rem)],
                        acc.at[pl.ds(sid * _RPT + full * _K, rem)])

    @pl.when(sid == _NS - 1)
    def _zero_tail():
        pltpu.sync_copy(sb0.at[pl.ds(0, _RTAIL)],
                        acc.at[pl.ds(_RPT * _NS, _RTAIL)])

    plsc.subcore_barrier()

    def _scale(av, gb, sb):
        @plsc.parallel_loop(0, _K // _VL)
        def _body(eb):
            a16 = av[pl.ds(eb * _VL, _VL)]
            for i in range(_VL):
                e = eb * _VL + i
                a = a16[i]
                for j in range(_D // _VL):
                    sl = pl.ds(j * _VL, _VL)
                    sb[e, sl] = gb[e, sl] * a

    def _edge_src(arr, c):
        return arr.at[pl.ds(ebase + pl.multiple_of(c * _K, _K), _K)]

    slots = ((gb0, sb0, sx0, rx0, av0, gsem0, ssem0, xsem0, rsem0, asem0),
             (gb1, sb1, sx1, rx1, av1, gsem1, ssem1, xsem1, rsem1, asem1))

    for b, (gb, sb, sx, rx, av, gsem, ssem, xsem, rsem, asem) in enumerate(slots):
        pltpu.sync_copy(_edge_src(s_hbm, b), sx)
        pltpu.async_copy(_edge_src(a_hbm, b), av, asem)
        pltpu.async_copy(h_hbm.at[sx], gb, gsem)

    def _group(g, carry):
        for b, (gb, sb, sx, rx, av, gsem, ssem, xsem, rsem, asem) in enumerate(slots):
            c = g * 2 + b
            has_next = (c + 2 <= _NCHUNK - 1) if b == 0 else None

            pltpu.make_async_copy(h_hbm.at[sx], gb, gsem).wait()

            @pl.when(g >= 1)
            def _drain_prev():
                pltpu.make_async_copy(sb, acc.at[rx], ssem).wait()

            pltpu.async_copy(_edge_src(r_hbm, c), rx, rsem)

            def _fetch_next():
                pltpu.async_copy(_edge_src(s_hbm, c + 2), sx, xsem)

            if b == 0:
                _fetch_next()
            else:
                pl.when(g <= (_NCHUNK - 4) // 2)(_fetch_next)

            pltpu.make_async_copy(_edge_src(a_hbm, c), av, asem).wait()
            _scale(av, gb, sb)

            def _fetch_next_attr():
                pltpu.async_copy(_edge_src(a_hbm, c + 2), av, asem)

            if b == 0:
                _fetch_next_attr()
            else:
                pl.when(g <= (_NCHUNK - 4) // 2)(_fetch_next_attr)

            pltpu.make_async_copy(_edge_src(r_hbm, c), rx, rsem).wait()
            pltpu.async_copy(sb, acc.at[rx], ssem, add=True)

            def _start_next_gather():
                pltpu.make_async_copy(_edge_src(s_hbm, c + 2), sx, xsem).wait()
                pltpu.async_copy(h_hbm.at[sx], gb, gsem)

            if b == 0:
                _start_next_gather()
            else:
                pl.when(g <= (_NCHUNK - 4) // 2)(_start_next_gather)
        return carry

    _ngroup = (_NCHUNK - 1) // 2
    lax.fori_loop(0, _ngroup, _group, None)

    tail = tuple(zip(range(2 * _ngroup, _NCHUNK), slots))
    for ct, (gb, sb, sx, rx, av, gsem, ssem, xsem, rsem, asem) in tail:
        pltpu.make_async_copy(h_hbm.at[sx], gb, gsem).wait()
        pltpu.make_async_copy(sb, acc.at[rx], ssem).wait()
        pltpu.sync_copy(_edge_src(r_hbm, ct), rx)
        pltpu.make_async_copy(_edge_src(a_hbm, ct), av, asem).wait()
        _scale(av, gb, sb)
        pltpu.async_copy(sb, acc.at[rx], ssem, add=True)
    for ct, (gb, sb, sx, rx, av, gsem, ssem, xsem, rsem, asem) in tail:
        pltpu.make_async_copy(sb, acc.at[rx], ssem).wait()
    for b, (gb, sb, sx, rx, av, gsem, ssem, xsem, rsem, asem) in enumerate(slots):
        if b >= _NCHUNK - 2 * _ngroup:
            pltpu.make_async_copy(sb, acc.at[rx], ssem).wait()

    plsc.subcore_barrier()
    pltpu.sync_copy(acc.at[pl.ds(sid * _RPT, _RPT)],
                    out_hbm.at[cid, pl.ds(sid * _RPT, _RPT)])

    @pl.when(sid == _NS - 1)
    def _write_tail():
        pltpu.sync_copy(acc.at[pl.ds(_RPT * _NS, _RTAIL)],
                        out_hbm.at[cid, pl.ds(_RPT * _NS, _RTAIL)])


_BN = 2000
_NB = _N // _BN


def _enc_body(x_ref, w0_ref, b0_ref, w1_ref, b1_ref, o_ref):
    h = jnp.dot(x_ref[...], w0_ref[...], preferred_element_type=jnp.float32)
    h = jnp.maximum(h + b0_ref[...], 0.0)
    h = jnp.dot(h, w1_ref[...], preferred_element_type=jnp.float32)
    o_ref[...] = jnp.maximum(h + b1_ref[...], 0.0)


def _hop_body(p_ref, w_ref, b_ref, o_ref):
    conv = p_ref[0] + p_ref[1]
    u = jnp.dot(conv, w_ref[...], preferred_element_type=jnp.float32)
    o_ref[...] = jnp.maximum(u + b_ref[...], 0.0) + conv


def _dec_body(h_ref, w0_ref, b0_ref, w1_ref, b1_ref, o_ref):
    u = jnp.dot(h_ref[...], w0_ref[...], preferred_element_type=jnp.float32)
    u = jnp.maximum(u + b0_ref[...], 0.0)
    u = jnp.dot(u, w1_ref[...], preferred_element_type=jnp.float32)
    o_ref[...] = u + b1_ref[...]


def _row_blocked(body, n_weight_args):
    w_specs = [pl.BlockSpec((_D, _D), lambda i: (0, 0)),
               pl.BlockSpec((1, _D), lambda i: (0, 0))] * n_weight_args
    return pl.pallas_call(
        body,
        grid=(_NB,),
        in_specs=[pl.BlockSpec((_BN, _D), lambda i: (i, 0))] + w_specs,
        out_specs=pl.BlockSpec((_BN, _D), lambda i: (i, 0)),
        out_shape=jax.ShapeDtypeStruct((_N, _D), jnp.float32),
    )


_encoder = _row_blocked(_enc_body, 2)
_decoder = _row_blocked(_dec_body, 2)

_hop = pl.pallas_call(
    _hop_body,
    grid=(_NB,),
    in_specs=[pl.BlockSpec((_NC, _BN, _D), lambda i: (0, i, 0)),
              pl.BlockSpec((_D, _D), lambda i: (0, 0)),
              pl.BlockSpec((1, _D), lambda i: (0, 0))],
    out_specs=pl.BlockSpec((_BN, _D), lambda i: (i, 0)),
    out_shape=jax.ShapeDtypeStruct((_N, _D), jnp.float32),
)


def kernel(x, edge_index, edge_attr, enc_W0, enc_b0, enc_W1, enc_b1,
           hop0_W, hop0_b, hop1_W, hop1_b, hop2_W, hop2_b,
           dec_W0, dec_b0, dec_W1, dec_b1):
    senders = edge_index[1]
    receivers = edge_index[0]
    attr = edge_attr[:, 0]

    h = _encoder(x, enc_W0, enc_b0.reshape(1, _D), enc_W1, enc_b1.reshape(1, _D))
    for W, b in ((hop0_W, hop0_b), (hop1_W, hop1_b), (hop2_W, hop2_b)):
        partials = _sc_conv(h, senders, receivers, attr)
        h = _hop(partials, W, b.reshape(1, _D))

    c = dec_W1.shape[1]
    w1p = jnp.pad(dec_W1, ((0, 0), (0, _D - c)))
    b1p = jnp.pad(dec_b1, (0, _D - c)).reshape(1, _D)
    out = _decoder(h, dec_W0, dec_b0.reshape(1, _D), w1p, b1p)
    return out[:, :c]


# speedup vs baseline: 1.0075x; 1.0024x over previous
import functools

import jax
import jax.numpy as jnp
from jax import lax
from jax.experimental import pallas as pl
from jax.experimental.pallas import tpu as pltpu
from jax.experimental.pallas import tpu_sc as plsc

_N, _E, _D = 10000, 320000, 128
_NC, _NS = 2, 16
_NW = _NC * _NS
_EPW = _E // _NW
_K = 80
_NCHUNK = _EPW // _K
_RPT = 624
_RTAIL = _N - _RPT * _NS
_VL = 16

_mesh = plsc.VectorSubcoreMesh(core_axis_name="c", subcore_axis_name="s")


@functools.partial(
    pl.kernel,
    out_type=jax.ShapeDtypeStruct((_NC, _N, _D), jnp.float32),
    mesh=_mesh,
    scratch_types=[
        pltpu.VMEM((_K,), jnp.int32),
        pltpu.VMEM((_K,), jnp.int32),
        pltpu.VMEM((_K,), jnp.int32),
        pltpu.VMEM((_K,), jnp.int32),
        pltpu.VMEM((_K,), jnp.float32),
        pltpu.VMEM((_K,), jnp.float32),
        pltpu.VMEM((_K, _D), jnp.float32),
        pltpu.VMEM((_K, _D), jnp.float32),
        pltpu.VMEM((_K, _D), jnp.float32),
        pltpu.VMEM((_K, _D), jnp.float32),
        pltpu.VMEM_SHARED((_N, _D), jnp.float32),
        pltpu.SemaphoreType.DMA,
        pltpu.SemaphoreType.DMA,
        pltpu.SemaphoreType.DMA,
        pltpu.SemaphoreType.DMA,
        pltpu.SemaphoreType.DMA,
        pltpu.SemaphoreType.DMA,
        pltpu.SemaphoreType.DMA,
        pltpu.SemaphoreType.DMA,
        pltpu.SemaphoreType.DMA,
        pltpu.SemaphoreType.DMA,
    ],
)
def _sc_conv(h_hbm, s_hbm, r_hbm, a_hbm, out_hbm,
             sx0, sx1, rx0, rx1, av0, av1, gb0, gb1, sb0, sb1, acc,
             gsem0, gsem1, ssem0, ssem1, xsem0, xsem1, rsem0, rsem1,
             asem0, asem1):
    cid = lax.axis_index("c")
    sid = lax.axis_index("s")
    wid = sid * _NC + cid
    ebase = wid * _EPW

    def _zfill(i, carry):
        r = i // (_D // _VL)
        j = lax.rem(i, _D // _VL)
        sb0[r, pl.ds(j * _VL, _VL)] = jnp.zeros((_VL,), jnp.float32)
        return carry

    lax.fori_loop(0, _K * (_D // _VL), _zfill, None)
    full, rem = _RPT // _K, _RPT % _K
    zcopies = [(sb0, acc.at[pl.ds(sid * _RPT + k * _K, _K)])
               for k in range(full)]
    if rem:
        zcopies.append((sb0.at[pl.ds(0, rem)],
                        acc.at[pl.ds(sid * _RPT + full * _K, rem)]))
    for src, dst in zcopies:
        pltpu.async_copy(src, dst, asem0)
    for src, dst in zcopies:
        pltpu.make_async_copy(src, dst, asem0).wait()

    @pl.when(sid == _NS - 1)
    def _zero_tail():
        pltpu.sync_copy(sb0.at[pl.ds(0, _RTAIL)],
                        acc.at[pl.ds(_RPT * _NS, _RTAIL)])

    plsc.subcore_barrier()

    def _scale(av, gb, sb):
        @plsc.parallel_loop(0, _K // _VL)
        def _body(eb):
            a16 = av[pl.ds(eb * _VL, _VL)]
            for i in range(_VL):
                e = eb * _VL + i
                a = a16[i]
                for j in range(_D // _VL):
                    sl = pl.ds(j * _VL, _VL)
                    sb[e, sl] = gb[e, sl] * a

    def _edge_src(arr, c):
        return arr.at[pl.ds(ebase + pl.multiple_of(c * _K, _K), _K)]

    slots = ((gb0, sb0, sx0, rx0, av0, gsem0, ssem0, xsem0, rsem0, asem0),
             (gb1, sb1, sx1, rx1, av1, gsem1, ssem1, xsem1, rsem1, asem1))

    for b, (gb, sb, sx, rx, av, gsem, ssem, xsem, rsem, asem) in enumerate(slots):
        pltpu.sync_copy(_edge_src(s_hbm, b), sx)
        pltpu.async_copy(_edge_src(a_hbm, b), av, asem)
        pltpu.async_copy(h_hbm.at[sx], gb, gsem)

    def _group(g, carry):
        for b, (gb, sb, sx, rx, av, gsem, ssem, xsem, rsem, asem) in enumerate(slots):
            c = g * 2 + b
            has_next = (c + 2 <= _NCHUNK - 1) if b == 0 else None

            pltpu.make_async_copy(h_hbm.at[sx], gb, gsem).wait()

            @pl.when(g >= 1)
            def _drain_prev():
                pltpu.make_async_copy(sb, acc.at[rx], ssem).wait()

            pltpu.async_copy(_edge_src(r_hbm, c), rx, rsem)

            def _fetch_next():
                pltpu.async_copy(_edge_src(s_hbm, c + 2), sx, xsem)

            if b == 0:
                _fetch_next()
            else:
                pl.when(g <= (_NCHUNK - 4) // 2)(_fetch_next)

            pltpu.make_async_copy(_edge_src(a_hbm, c), av, asem).wait()
            _scale(av, gb, sb)

            def _fetch_next_attr():
                pltpu.async_copy(_edge_src(a_hbm, c + 2), av, asem)

            if b == 0:
                _fetch_next_attr()
            else:
                pl.when(g <= (_NCHUNK - 4) // 2)(_fetch_next_attr)

            pltpu.make_async_copy(_edge_src(r_hbm, c), rx, rsem).wait()
            pltpu.async_copy(sb, acc.at[rx], ssem, add=True)

            def _start_next_gather():
                pltpu.make_async_copy(_edge_src(s_hbm, c + 2), sx, xsem).wait()
                pltpu.async_copy(h_hbm.at[sx], gb, gsem)

            if b == 0:
                _start_next_gather()
            else:
                pl.when(g <= (_NCHUNK - 4) // 2)(_start_next_gather)
        return carry

    _ngroup = (_NCHUNK - 1) // 2
    lax.fori_loop(0, _ngroup, _group, None)

    tail = tuple(zip(range(2 * _ngroup, _NCHUNK), slots))
    for ct, (gb, sb, sx, rx, av, gsem, ssem, xsem, rsem, asem) in tail:
        pltpu.make_async_copy(h_hbm.at[sx], gb, gsem).wait()
        pltpu.make_async_copy(sb, acc.at[rx], ssem).wait()
        pltpu.sync_copy(_edge_src(r_hbm, ct), rx)
        pltpu.make_async_copy(_edge_src(a_hbm, ct), av, asem).wait()
        _scale(av, gb, sb)
        pltpu.async_copy(sb, acc.at[rx], ssem, add=True)
    for ct, (gb, sb, sx, rx, av, gsem, ssem, xsem, rsem, asem) in tail:
        pltpu.make_async_copy(sb, acc.at[rx], ssem).wait()
    for b, (gb, sb, sx, rx, av, gsem, ssem, xsem, rsem, asem) in enumerate(slots):
        if b >= _NCHUNK - 2 * _ngroup:
            pltpu.make_async_copy(sb, acc.at[rx], ssem).wait()

    plsc.subcore_barrier()
    pltpu.sync_copy(acc.at[pl.ds(sid * _RPT, _RPT)],
                    out_hbm.at[cid, pl.ds(sid * _RPT, _RPT)])

    @pl.when(sid == _NS - 1)
    def _write_tail():
        pltpu.sync_copy(acc.at[pl.ds(_RPT * _NS, _RTAIL)],
                        out_hbm.at[cid, pl.ds(_RPT * _NS, _RTAIL)])


_BN = 2000
_NB = _N // _BN


def _enc_body(x_ref, w0_ref, b0_ref, w1_ref, b1_ref, o_ref):
    h = jnp.dot(x_ref[...], w0_ref[...], preferred_element_type=jnp.float32)
    h = jnp.maximum(h + b0_ref[...], 0.0)
    h = jnp.dot(h, w1_ref[...], preferred_element_type=jnp.float32)
    o_ref[...] = jnp.maximum(h + b1_ref[...], 0.0)


def _hop_body(p_ref, w_ref, b_ref, o_ref):
    conv = p_ref[0] + p_ref[1]
    u = jnp.dot(conv, w_ref[...], preferred_element_type=jnp.float32)
    o_ref[...] = jnp.maximum(u + b_ref[...], 0.0) + conv


def _dec_body(h_ref, w0_ref, b0_ref, w1_ref, b1_ref, o_ref):
    u = jnp.dot(h_ref[...], w0_ref[...], preferred_element_type=jnp.float32)
    u = jnp.maximum(u + b0_ref[...], 0.0)
    u = jnp.dot(u, w1_ref[...], preferred_element_type=jnp.float32)
    o_ref[...] = u + b1_ref[...]


def _row_blocked(body, n_weight_args):
    w_specs = [pl.BlockSpec((_D, _D), lambda i: (0, 0)),
               pl.BlockSpec((1, _D), lambda i: (0, 0))] * n_weight_args
    return pl.pallas_call(
        body,
        grid=(_NB,),
        in_specs=[pl.BlockSpec((_BN, _D), lambda i: (i, 0))] + w_specs,
        out_specs=pl.BlockSpec((_BN, _D), lambda i: (i, 0)),
        out_shape=jax.ShapeDtypeStruct((_N, _D), jnp.float32),
    )


_encoder = _row_blocked(_enc_body, 2)
_decoder = _row_blocked(_dec_body, 2)

_hop = pl.pallas_call(
    _hop_body,
    grid=(_NB,),
    in_specs=[pl.BlockSpec((_NC, _BN, _D), lambda i: (0, i, 0)),
              pl.BlockSpec((_D, _D), lambda i: (0, 0)),
              pl.BlockSpec((1, _D), lambda i: (0, 0))],
    out_specs=pl.BlockSpec((_BN, _D), lambda i: (i, 0)),
    out_shape=jax.ShapeDtypeStruct((_N, _D), jnp.float32),
)


def kernel(x, edge_index, edge_attr, enc_W0, enc_b0, enc_W1, enc_b1,
           hop0_W, hop0_b, hop1_W, hop1_b, hop2_W, hop2_b,
           dec_W0, dec_b0, dec_W1, dec_b1):
    senders = edge_index[1]
    receivers = edge_index[0]
    attr = edge_attr[:, 0]

    h = _encoder(x, enc_W0, enc_b0.reshape(1, _D), enc_W1, enc_b1.reshape(1, _D))
    for W, b in ((hop0_W, hop0_b), (hop1_W, hop1_b), (hop2_W, hop2_b)):
        partials = _sc_conv(h, senders, receivers, attr)
        h = _hop(partials, W, b.reshape(1, _D))

    c = dec_W1.shape[1]
    w1p = jnp.pad(dec_W1, ((0, 0), (0, _D - c)))
    b1p = jnp.pad(dec_b1, (0, _D - c)).reshape(1, _D)
    out = _decoder(h, dec_W0, dec_b0.reshape(1, _D), w1p, b1p)
    return out[:, :c]
